# Initial kernel scaffold; baseline (speedup 1.0000x reference)
import jax
import jax.numpy as jnp
from jax.experimental import pallas as pl


def kernel(x, x_0, edge_index, edge_weight, W1, node_lock):
    raise NotImplementedError("write your pallas kernel here")


# speedup vs baseline: 3.4180x; 3.4180x over previous
import functools
from math import log

import jax
import jax.numpy as jnp
from jax import lax
from jax.experimental import pallas as pl
from jax.experimental.pallas import tpu as pltpu
from jax.experimental.pallas import tpu_sc as plsc

ALPHA = 0.1
BETA = log(0.5 / (2 + 1) + 1)

NC = 2
NS = 16
L = 16

K = 128


def _sc_segment_sum(x2, src, dst, w, N):
    E = src.shape[0]
    num_chunks = E // K
    assert E % K == 0
    G = 80
    num_groups = N // G
    assert N % G == 0
    mesh = plsc.VectorSubcoreMesh(core_axis_name="c", subcore_axis_name="s")

    @functools.partial(
        pl.kernel,
        mesh=mesh,
        out_type=jax.ShapeDtypeStruct((NC, N, 128), jnp.float32),
        scratch_types=[
            pltpu.VMEM((K,), jnp.int32),
            pltpu.VMEM((K,), jnp.int32),
            pltpu.VMEM((K,), jnp.int32),
            pltpu.VMEM((K,), jnp.float32),
            pltpu.VMEM((K, 128), jnp.float32),
            pltpu.VMEM_SHARED((N, 128), jnp.float32),
            pltpu.SemaphoreType.DMA,
        ],
    )
    def seg_sum(x2_hbm, src_hbm, dst_hbm, w_hbm, out_hbm,
                sidx_v, gidx_v, didx_v, w_v, rows_v, acc_sh, sem):
        c = lax.axis_index("c")
        s = lax.axis_index("s")

        zero16 = jnp.zeros((L,), jnp.float32)

        @pl.loop(0, K)
        def _zero_rows(r):
            for j in range(128 // L):
                rows_v[r, pl.ds(j * L, L)] = zero16

        @pl.loop(s, num_groups, step=NS)
        def _zero_acc(g):
            pltpu.sync_copy(rows_v.at[pl.ds(0, G)], acc_sh.at[pl.ds(g * G, G)])

        plsc.subcore_barrier()

        @pl.loop(s, num_chunks, step=NS)
        def _chunk(cid):
            ebase = cid * K
            pltpu.sync_copy(src_hbm.at[pl.ds(ebase, K)], sidx_v)
            pltpu.sync_copy(dst_hbm.at[pl.ds(ebase, K)], didx_v)
            pltpu.sync_copy(w_hbm.at[pl.ds(ebase, K)], w_v)
            for i in range(K // L):
                sl = pl.ds(i * L, L)
                gidx_v[sl] = sidx_v[sl] * 2 + c
            pltpu.async_copy(x2_hbm.at[gidx_v], rows_v, sem).wait()

            @pl.loop(0, K, step=L)
            def _wblk(b):
                wv = w_v[pl.ds(b, L)]
                dnums = lax.GatherDimensionNumbers(
                    offset_dims=(), collapsed_slice_dims=(0,),
                    start_index_map=(0,))
                for e in range(L):
                    wvec = lax.gather(
                        wv, jnp.full((L, 1), e, jnp.int32), dnums,
                        slice_sizes=(1,),
                        mode=lax.GatherScatterMode.PROMISE_IN_BOUNDS)
                    for j in range(128 // L):
                        sl = pl.ds(j * L, L)
                        rows_v[b + e, sl] = rows_v[b + e, sl] * wvec

            pltpu.sync_copy(rows_v, acc_sh.at[didx_v], add=True)

        plsc.subcore_barrier()

        @pl.loop(s, num_groups, step=NS)
        def _copy_out(g):
            pltpu.sync_copy(acc_sh.at[pl.ds(g * G, G)],
                            out_hbm.at[c, pl.ds(g * G, G)])

    return seg_sum(x2, src, dst, w)


def _tc_combine(a0, a1, x_0, W1):
    N, C = x_0.shape
    TN = 400
    assert N % TN == 0

    def body(a0_ref, a1_ref, x0_ref, w_ref, out_ref):
        h = jnp.concatenate([a0_ref[...], a1_ref[...]], axis=-1)
        h = h * (1.0 - ALPHA) + ALPHA * x0_ref[...]
        out_ref[...] = (1.0 - BETA) * h + BETA * jnp.dot(
            h, w_ref[...], preferred_element_type=jnp.float32)

    return pl.pallas_call(
        body,
        grid=(N // TN,),
        in_specs=[
            pl.BlockSpec((TN, C // 2), lambda i: (i, 0)),
            pl.BlockSpec((TN, C // 2), lambda i: (i, 0)),
            pl.BlockSpec((TN, C), lambda i: (i, 0)),
            pl.BlockSpec((C, C), lambda i: (0, 0)),
        ],
        out_specs=pl.BlockSpec((TN, C), lambda i: (i, 0)),
        out_shape=jax.ShapeDtypeStruct((N, C), jnp.float32),
    )(a0, a1, x_0, W1)


def kernel(x, x_0, edge_index, edge_weight, W1, node_lock):
    N, C = x.shape
    assert C == 256
    x2 = x.reshape(2 * N, 128)
    src = edge_index[0]
    dst = edge_index[1]
    agg = _sc_segment_sum(x2, src, dst, edge_weight, N)
    return _tc_combine(agg[0], agg[1], x_0, W1)


# speedup vs baseline: 6.5671x; 1.9214x over previous
import functools
from math import log

import jax
import jax.numpy as jnp
from jax import lax
from jax.experimental import pallas as pl
from jax.experimental.pallas import tpu as pltpu
from jax.experimental.pallas import tpu_sc as plsc

ALPHA = 0.1
BETA = log(0.5 / (2 + 1) + 1)

NC = 2
NS = 16
L = 16

K = 80


_DNUMS = lax.GatherDimensionNumbers(
    offset_dims=(), collapsed_slice_dims=(0,), start_index_map=(0,))


def _sc_segment_sum(x2, src, dst, w, N):
    E = src.shape[0]
    ept = E // NS
    nch = ept // K
    assert E % NS == 0 and ept % K == 0 and K % 8 == 0 and (ept // K) % 2 == 1
    num_groups = N // K
    assert N % K == 0
    mesh = plsc.VectorSubcoreMesh(core_axis_name="c", subcore_axis_name="s")

    @functools.partial(
        pl.kernel,
        mesh=mesh,
        out_type=jax.ShapeDtypeStruct((NC, N, 128), jnp.float32),
        scratch_types=[
            pltpu.VMEM((ept,), jnp.int32),
            pltpu.VMEM((ept,), jnp.float32),
            pltpu.VMEM((K,), jnp.int32),
            pltpu.VMEM((K,), jnp.int32),
            pltpu.VMEM((K,), jnp.int32),
            pltpu.VMEM((K,), jnp.int32),
            pltpu.VMEM((K,), jnp.int32),
            pltpu.VMEM((K, 128), jnp.float32),
            pltpu.VMEM((K, 128), jnp.float32),
            pltpu.VMEM_SHARED((N, 128), jnp.float32),
            pltpu.SemaphoreType.DMA,
            pltpu.SemaphoreType.DMA,
            pltpu.SemaphoreType.DMA,
            pltpu.SemaphoreType.DMA,
        ],
    )
    def seg_sum(x2_hbm, src_hbm, dst_hbm, w_hbm, out_hbm,
                didx_all, w_all, sidx0, sidx1, gidx0, gidx1, didx_v,
                rows0, rows1, acc_sh, gsem0, gsem1, csem0, csem1):
        c = lax.axis_index("c")
        s = lax.axis_index("s")
        sidx = (sidx0, sidx1)
        gidx = (gidx0, gidx1)
        rows = (rows0, rows1)
        gsem = (gsem0, gsem1)
        csem = (csem0, csem1)
        ebase = s * ept

        pltpu.sync_copy(dst_hbm.at[pl.ds(ebase, ept)], didx_all)
        pltpu.sync_copy(w_hbm.at[pl.ds(ebase, ept)], w_all)

        zero16 = jnp.zeros((L,), jnp.float32)

        @pl.loop(0, K)
        def _zero_rows(r):
            for j in range(128 // L):
                rows0[r, pl.ds(j * L, L)] = zero16

        @pl.loop(s, num_groups, step=NS)
        def _zero_acc(g):
            pltpu.sync_copy(rows0, acc_sh.at[pl.ds(g * K, K)])

        plsc.subcore_barrier()

        def issue_src(j, b):
            pltpu.async_copy(src_hbm.at[pl.ds(ebase + j * K, K)],
                             sidx[b], csem[b])

        def wait_src(j, b):
            pltpu.make_async_copy(src_hbm.at[pl.ds(ebase + j * K, K)],
                                  sidx[b], csem[b]).wait()

        def issue_gather(j, b):
            for i in range(K // L):
                sl = pl.ds(i * L, L)
                gidx[b][sl] = sidx[b][sl] * 2 + c
            pltpu.async_copy(x2_hbm.at[gidx[b]], rows[b], gsem[b])

        def chunk(j, b, issue_src2=True, issue_next=True):
            off = j * K
            b2 = 1 - b
            if issue_src2:
                issue_src(j + 2, b)
            if issue_next:
                wait_src(j + 1, b2)
                issue_gather(j + 1, b2)
            pltpu.make_async_copy(x2_hbm.at[gidx[b]], rows[b], gsem[b]).wait()

            @pl.loop(0, K, step=L)
            def _wblk(bb):
                wv = w_all[pl.ds(off + bb, L)]
                for e in range(L):
                    wvec = lax.gather(
                        wv, jnp.full((L, 1), e, jnp.int32), _DNUMS,
                        slice_sizes=(1,),
                        mode=lax.GatherScatterMode.PROMISE_IN_BOUNDS)
                    for jj in range(128 // L):
                        sl = pl.ds(jj * L, L)
                        rows[b][bb + e, sl] = rows[b][bb + e, sl] * wvec

            for i in range(K // L):
                sl = pl.ds(i * L, L)
                didx_v[sl] = didx_all[pl.ds(off + i * L, L)]
            pltpu.sync_copy(rows[b], acc_sh.at[didx_v], add=True)

        issue_src(jnp.int32(0), 0)
        wait_src(jnp.int32(0), 0)
        issue_gather(jnp.int32(0), 0)
        issue_src(jnp.int32(1), 1)

        @pl.loop(0, nch - 3, step=2)
        def _main(j):
            chunk(j, 0)
            chunk(j + 1, 1)

        chunk(jnp.int32(nch - 3), 0)
        chunk(jnp.int32(nch - 2), 1, issue_src2=False)
        chunk(jnp.int32(nch - 1), 0, issue_src2=False, issue_next=False)
        plsc.subcore_barrier()

        @pl.loop(s, num_groups, step=NS)
        def _copy_out(g):
            pltpu.sync_copy(acc_sh.at[pl.ds(g * K, K)],
                            out_hbm.at[c, pl.ds(g * K, K)])

    return seg_sum(x2, src, dst, w)


def _tc_combine(a0, a1, x_0, W1):
    N, C = x_0.shape
    TN = 400
    assert N % TN == 0

    def body(a0_ref, a1_ref, x0_ref, w_ref, out_ref):
        h = jnp.concatenate([a0_ref[...], a1_ref[...]], axis=-1)
        h = h * (1.0 - ALPHA) + ALPHA * x0_ref[...]
        out_ref[...] = (1.0 - BETA) * h + BETA * jnp.dot(
            h, w_ref[...], preferred_element_type=jnp.float32)

    return pl.pallas_call(
        body,
        grid=(N // TN,),
        in_specs=[
            pl.BlockSpec((TN, C // 2), lambda i: (i, 0)),
            pl.BlockSpec((TN, C // 2), lambda i: (i, 0)),
            pl.BlockSpec((TN, C), lambda i: (i, 0)),
            pl.BlockSpec((C, C), lambda i: (0, 0)),
        ],
        out_specs=pl.BlockSpec((TN, C), lambda i: (i, 0)),
        out_shape=jax.ShapeDtypeStruct((N, C), jnp.float32),
    )(a0, a1, x_0, W1)


def kernel(x, x_0, edge_index, edge_weight, W1, node_lock):
    N, C = x.shape
    assert C == 256
    x2 = x.reshape(2 * N, 128)
    src = edge_index[0]
    dst = edge_index[1]
    agg = _sc_segment_sum(x2, src, dst, edge_weight, N)
    return _tc_combine(agg[0], agg[1], x_0, W1)


# speedup vs baseline: 6.8023x; 1.0358x over previous
import functools
from math import log

import jax
import jax.numpy as jnp
from jax import lax
from jax.experimental import pallas as pl
from jax.experimental.pallas import tpu as pltpu
from jax.experimental.pallas import tpu_sc as plsc

ALPHA = 0.1
BETA = log(0.5 / (2 + 1) + 1)

NC = 2
NS = 16
L = 16

K = 80


_DNUMS = lax.GatherDimensionNumbers(
    offset_dims=(), collapsed_slice_dims=(0,), start_index_map=(0,))


def _sc_segment_sum(x2, src, dst, w, N):
    E = src.shape[0]
    ept = E // NS
    nch = ept // K
    assert E % NS == 0 and ept % K == 0 and K % 8 == 0 and (ept // K) % 2 == 1
    num_groups = N // K
    assert N % K == 0
    mesh = plsc.VectorSubcoreMesh(core_axis_name="c", subcore_axis_name="s")

    @functools.partial(
        pl.kernel,
        mesh=mesh,
        out_type=jax.ShapeDtypeStruct((N, NC * 128), jnp.float32),
        scratch_types=[
            pltpu.VMEM((ept,), jnp.int32),
            pltpu.VMEM((ept,), jnp.float32),
            pltpu.VMEM((K,), jnp.int32),
            pltpu.VMEM((K,), jnp.int32),
            pltpu.VMEM((K,), jnp.int32),
            pltpu.VMEM((K,), jnp.int32),
            pltpu.VMEM((K,), jnp.int32),
            pltpu.VMEM((K,), jnp.int32),
            pltpu.VMEM((K, 128), jnp.float32),
            pltpu.VMEM((K, 128), jnp.float32),
            pltpu.VMEM_SHARED((N, 128), jnp.float32),
            pltpu.SemaphoreType.DMA,
            pltpu.SemaphoreType.DMA,
            pltpu.SemaphoreType.DMA,
            pltpu.SemaphoreType.DMA,
            pltpu.SemaphoreType.DMA,
            pltpu.SemaphoreType.DMA,
        ],
    )
    def seg_sum(x2_hbm, src_hbm, dst_hbm, w_hbm, out_hbm,
                didx_all, w_all, sidx0, sidx1, gidx0, gidx1, didx0, didx1,
                rows0, rows1, acc_sh, gsem0, gsem1, csem0, csem1,
                ssem0, ssem1):
        c = lax.axis_index("c")
        s = lax.axis_index("s")
        sidx = (sidx0, sidx1)
        gidx = (gidx0, gidx1)
        didx = (didx0, didx1)
        rows = (rows0, rows1)
        gsem = (gsem0, gsem1)
        csem = (csem0, csem1)
        ssem = (ssem0, ssem1)
        ebase = s * ept

        pltpu.sync_copy(dst_hbm.at[pl.ds(ebase, ept)], didx_all)
        pltpu.sync_copy(w_hbm.at[pl.ds(ebase, ept)], w_all)

        zero16 = jnp.zeros((L,), jnp.float32)

        @pl.loop(0, K)
        def _zero_rows(r):
            for j in range(128 // L):
                rows0[r, pl.ds(j * L, L)] = zero16

        @pl.loop(s, num_groups, step=NS)
        def _zero_acc(g):
            pltpu.sync_copy(rows0, acc_sh.at[pl.ds(g * K, K)])

        plsc.subcore_barrier()

        def issue_src(j, b):
            pltpu.async_copy(src_hbm.at[pl.ds(ebase + j * K, K)],
                             sidx[b], csem[b])

        def wait_src(j, b):
            pltpu.make_async_copy(src_hbm.at[pl.ds(ebase + j * K, K)],
                                  sidx[b], csem[b]).wait()

        def issue_gather(j, b):
            for i in range(K // L):
                sl = pl.ds(i * L, L)
                gidx[b][sl] = sidx[b][sl] * 2 + c
            pltpu.async_copy(x2_hbm.at[gidx[b]], rows[b], gsem[b])

        def wait_scatter(b):
            pltpu.make_async_copy(rows[b], acc_sh.at[didx[b]], ssem[b]).wait()

        def chunk(j, b, first=False, issue_src2=True, issue_next=True):
            off = j * K
            b2 = 1 - b
            if issue_src2:
                issue_src(j + 2, b)
            if not first:
                wait_scatter(b2)
            if issue_next:
                wait_src(j + 1, b2)
                issue_gather(j + 1, b2)
            pltpu.make_async_copy(x2_hbm.at[gidx[b]], rows[b], gsem[b]).wait()

            @pl.loop(0, K, step=L)
            def _wblk(bb):
                wv = w_all[pl.ds(off + bb, L)]
                for e in range(L):
                    wvec = lax.gather(
                        wv, jnp.full((L, 1), e, jnp.int32), _DNUMS,
                        slice_sizes=(1,),
                        mode=lax.GatherScatterMode.PROMISE_IN_BOUNDS)
                    for jj in range(128 // L):
                        sl = pl.ds(jj * L, L)
                        rows[b][bb + e, sl] = rows[b][bb + e, sl] * wvec

            for i in range(K // L):
                sl = pl.ds(i * L, L)
                didx[b][sl] = didx_all[pl.ds(off + i * L, L)]
            pltpu.async_copy(rows[b], acc_sh.at[didx[b]], ssem[b], add=True)

        issue_src(jnp.int32(0), 0)
        wait_src(jnp.int32(0), 0)
        issue_gather(jnp.int32(0), 0)
        issue_src(jnp.int32(1), 1)

        chunk(jnp.int32(0), 0, first=True)

        @pl.loop(1, nch - 2, step=2)
        def _main(j):
            chunk(j, 1)
            chunk(j + 1, 0)

        chunk(jnp.int32(nch - 2), 1, issue_src2=False)
        chunk(jnp.int32(nch - 1), 0, issue_src2=False, issue_next=False)
        wait_scatter(0)
        plsc.subcore_barrier()

        col = pl.multiple_of(c * 128, 128)

        @pl.loop(s, num_groups, step=NS)
        def _copy_out(g):
            pltpu.sync_copy(acc_sh.at[pl.ds(g * K, K)],
                            out_hbm.at[pl.ds(g * K, K), pl.ds(col, 128)])

    return seg_sum(x2, src, dst, w)


def _tc_combine(agg, x_0, W1):
    N, C = x_0.shape
    TN = 400
    assert N % TN == 0

    def body(a_ref, x0_ref, w_ref, out_ref):
        h = a_ref[...] * (1.0 - ALPHA) + ALPHA * x0_ref[...]
        out_ref[...] = (1.0 - BETA) * h + BETA * jnp.dot(
            h, w_ref[...], preferred_element_type=jnp.float32)

    return pl.pallas_call(
        body,
        grid=(N // TN,),
        in_specs=[
            pl.BlockSpec((TN, C), lambda i: (i, 0)),
            pl.BlockSpec((TN, C), lambda i: (i, 0)),
            pl.BlockSpec((C, C), lambda i: (0, 0)),
        ],
        out_specs=pl.BlockSpec((TN, C), lambda i: (i, 0)),
        out_shape=jax.ShapeDtypeStruct((N, C), jnp.float32),
    )(agg, x_0, W1)


def kernel(x, x_0, edge_index, edge_weight, W1, node_lock):
    N, C = x.shape
    assert C == 256
    x2 = x.reshape(2 * N, 128)
    src = edge_index[0]
    dst = edge_index[1]
    agg = _sc_segment_sum(x2, src, dst, edge_weight, N)
    return _tc_combine(agg, x_0, W1)
